# baseline (speedup 1.0000x reference)
import functools

import jax
import jax.numpy as jnp
from jax import lax
from jax.experimental import pallas as pl
from jax.experimental.pallas import tpu as pltpu
from jax.experimental.pallas import tpu_sc as plsc

N = 8192

_info = plsc.get_sparse_core_info()
_NC, _NS, _L = _info.num_cores, _info.num_subcores, _info.num_lanes
_NW = _NC * _NS
_RPW = N // _NW
_RC = 8

_mesh = plsc.VectorSubcoreMesh(core_axis_name="c", subcore_axis_name="s")


@functools.partial(
    pl.kernel,
    mesh=_mesh,
    out_type=jax.ShapeDtypeStruct((N, N), jnp.float32),
    scratch_types=[
        pltpu.VMEM((N,), jnp.float32),
        pltpu.VMEM((_RC, N), jnp.float32),
    ],
)
def _bt_sc(abil_hbm, out_hbm, abil_v, buf_v):
    wid = lax.axis_index("s") * _NC + lax.axis_index("c")
    pltpu.sync_copy(abil_hbm, abil_v)
    base = wid * _RPW

    def group_body(g, _):
        row0 = base + g * _L
        rv = abil_v[pl.ds(row0, _L)]
        for half in range(_L // _RC):
            for rr in range(_RC):
                b = jnp.full((_L,), rv[half * _RC + rr], jnp.float32)

                def jbody(j, _, rr=rr, b=b):
                    v = abil_v[pl.ds(j * _L, _L)]
                    buf_v[rr, pl.ds(j * _L, _L)] = 1.0 / (1.0 + jnp.exp(v - b))
                    return 0

                lax.fori_loop(0, N // _L, jbody, 0, unroll=8)
            pltpu.sync_copy(buf_v, out_hbm.at[pl.ds(row0 + half * _RC, _RC)])
        return 0

    lax.fori_loop(0, _RPW // _L, group_body, 0)


def kernel(ability):
    return _bt_sc(ability)


# speedup vs baseline: 5.4131x; 5.4131x over previous
import functools

import jax
import jax.numpy as jnp
from jax import lax
from jax.experimental import pallas as pl
from jax.experimental.pallas import tpu as pltpu
from jax.experimental.pallas import tpu_sc as plsc

N = 8192

_info = plsc.get_sparse_core_info()
_NC, _NS, _L = _info.num_cores, _info.num_subcores, _info.num_lanes
_NW = _NC * _NS
_RPW = N // _NW
_RC = 8

_mesh = plsc.VectorSubcoreMesh(core_axis_name="c", subcore_axis_name="s")


@functools.partial(
    pl.kernel,
    mesh=_mesh,
    out_type=jax.ShapeDtypeStruct((N, N), jnp.float32),
    scratch_types=[
        pltpu.VMEM((N,), jnp.float32),
        pltpu.VMEM((_RC, N), jnp.float32),
    ],
)
def _bt_sc(abil_hbm, out_hbm, abil_v, buf_v):
    wid = lax.axis_index("s") * _NC + lax.axis_index("c")
    pltpu.sync_copy(abil_hbm, abil_v)
    base = wid * _RPW

    def group_body(g, _):
        row0 = base + g * _L
        rv = abil_v[pl.ds(row0, _L)]
        for half in range(_L // _RC):
            bs = [jnp.full((_L,), rv[half * _RC + rr], jnp.float32)
                  for rr in range(_RC)]

            def jbody(j, _, bs=bs):
                v = abil_v[pl.ds(j * _L, _L)]
                for rr in range(_RC):
                    buf_v[rr, pl.ds(j * _L, _L)] = (
                        1.0 / (1.0 + jnp.exp(v - bs[rr])))
                return 0

            lax.fori_loop(0, N // _L, jbody, 0, unroll=2)
            pltpu.sync_copy(buf_v, out_hbm.at[pl.ds(row0 + half * _RC, _RC)])
        return 0

    lax.fori_loop(0, _RPW // _L, group_body, 0)


def kernel(ability):
    return _bt_sc(ability)
